# baseline (speedup 1.0000x reference)
import jax
import jax.numpy as jnp
from jax import lax
from jax.experimental import pallas as pl
from jax.experimental.pallas import tpu as pltpu

_INITIAL_VOCAB = 100000
_LANES = 256
_CH = 1800
_NCH = 125
_K = 12


def _body(idx_smem, emb_in, usage_in, ctime_in, nemb_in,
          emb_out, usage_out, ctime_out, row_out,
          bufs, ubuf, cbuf, nbuf,
          sem_in, sem_out, sem_u, sem_c, sem_n):
    tok = idx_smem[0]
    exp_row = tok - _INITIAL_VOCAB
    fr_all = exp_row // 4
    sub = exp_row % 4

    u_in = pltpu.make_async_copy(usage_in, ubuf, sem_u)
    u_in.start()
    c_in = pltpu.make_async_copy(ctime_in, cbuf, sem_c)
    c_in.start()
    n_in = pltpu.make_async_copy(nemb_in, nbuf, sem_n)
    n_in.start()

    in_cps = {}
    out_cps = {}
    for c in range(_K):
        cp = pltpu.make_async_copy(
            emb_in.at[pl.ds(c * _CH, _CH)], bufs.at[c % _K],
            sem_in.at[c % _K])
        cp.start()
        in_cps[c] = cp

    n_in.wait()
    row_cp = pltpu.make_async_copy(nbuf, row_out, sem_n)
    row_cp.start()

    lane = lax.broadcasted_iota(jnp.int32, (1, _LANES), 1)
    hit_lanes = (lane >= sub * 64) & (lane < sub * 64 + 64)

    for c in range(_NCH):
        q = c % _K
        in_cps[c].wait()

        local = fr_all - c * _CH

        @pl.when((local >= 0) & (local < _CH))
        def _(q=q, local=local):
            r = bufs.at[q][pl.ds(local, 1), :]
            bufs.at[q][pl.ds(local, 1), :] = jnp.where(hit_lanes,
                                                       nbuf[...], r)

        cp = pltpu.make_async_copy(
            bufs.at[q], emb_out.at[pl.ds(c * _CH, _CH)], sem_out.at[q])
        cp.start()
        out_cps[c] = cp
        nxt = c + _K
        if nxt < _NCH:
            out_cps[c].wait()
            cp2 = pltpu.make_async_copy(
                emb_in.at[pl.ds(nxt * _CH, _CH)], bufs.at[q], sem_in.at[q])
            cp2.start()
            in_cps[nxt] = cp2

    ur = tok // 64
    uc = tok % 64
    lane64 = lax.broadcasted_iota(jnp.int32, (1, 64), 1)
    u_in.wait()
    urow = ubuf[pl.ds(ur, 1), :]
    ubuf[pl.ds(ur, 1), :] = urow + (lane64 == uc).astype(jnp.float32)
    u_out = pltpu.make_async_copy(ubuf, usage_out, sem_u)
    u_out.start()

    c_in.wait()
    crow = cbuf[pl.ds(ur, 1), :]
    cbuf[pl.ds(ur, 1), :] = jnp.where(lane64 == uc, jnp.float32(jnp.inf),
                                      crow)
    c_out = pltpu.make_async_copy(cbuf, ctime_out, sem_c)
    c_out.start()

    for c in range(_NCH - _K, _NCH):
        out_cps[c].wait()
    row_cp.wait()
    u_out.wait()
    c_out.wait()


def kernel(token_usage, token_creation_time, expanded_embeddings,
           new_embedding, new_token_id):
    idx = jnp.asarray(new_token_id, jnp.int32).reshape(1)
    n_rows, dim = expanded_embeddings.shape
    emb4 = expanded_embeddings.reshape(-1, _LANES)
    nemb4 = jnp.tile(new_embedding, _LANES // dim).reshape(1, _LANES)
    usage2 = token_usage.reshape(-1, 64)
    ctime2 = token_creation_time.reshape(-1, 64)

    expanded, usage, ctime, row = pl.pallas_call(
        _body,
        in_specs=[
            pl.BlockSpec(memory_space=pltpu.SMEM),
            pl.BlockSpec(memory_space=pl.ANY),
            pl.BlockSpec(memory_space=pl.ANY),
            pl.BlockSpec(memory_space=pl.ANY),
            pl.BlockSpec(memory_space=pl.ANY),
        ],
        out_specs=[
            pl.BlockSpec(memory_space=pl.ANY),
            pl.BlockSpec(memory_space=pl.ANY),
            pl.BlockSpec(memory_space=pl.ANY),
            pl.BlockSpec(memory_space=pl.ANY),
        ],
        out_shape=[
            jax.ShapeDtypeStruct(emb4.shape, jnp.float32),
            jax.ShapeDtypeStruct(usage2.shape, jnp.float32),
            jax.ShapeDtypeStruct(ctime2.shape, jnp.float32),
            jax.ShapeDtypeStruct((1, _LANES), jnp.float32),
        ],
        scratch_shapes=[
            pltpu.VMEM((_K, _CH, _LANES), jnp.float32),
            pltpu.VMEM(usage2.shape, jnp.float32),
            pltpu.VMEM(ctime2.shape, jnp.float32),
            pltpu.VMEM((1, _LANES), jnp.float32),
            pltpu.SemaphoreType.DMA((_K,)),
            pltpu.SemaphoreType.DMA((_K,)),
            pltpu.SemaphoreType.DMA,
            pltpu.SemaphoreType.DMA,
            pltpu.SemaphoreType.DMA,
        ],
    )(idx, emb4, usage2, ctime2, nemb4)
    return (row.reshape(-1)[:dim], expanded.reshape(n_rows, dim),
            usage.reshape(-1), ctime.reshape(-1))


# speedup vs baseline: 1.0290x; 1.0290x over previous
import jax
import jax.numpy as jnp
from jax import lax
from jax.experimental import pallas as pl
from jax.experimental.pallas import tpu as pltpu

_INITIAL_VOCAB = 100000
_E = 57600000
_NW = 15
_W = _E // _NW
_SUB = 5
_SS = _W // _SUB


def _body(idx_smem, emb_in, usage_in, ctime_in, nemb_in,
          emb_out, usage_out, ctime_out, row_out,
          bufs, ubuf, cbuf, nbuf, tbuf,
          sem_in, sem_out, sem_u, sem_c, sem_n, sem_t):
    tok = idx_smem[0]
    exp_row = tok - _INITIAL_VOCAB

    u_in = pltpu.make_async_copy(usage_in, ubuf, sem_u)
    u_in.start()
    c_in = pltpu.make_async_copy(ctime_in, cbuf, sem_c)
    c_in.start()
    n_in = pltpu.make_async_copy(nemb_in, nbuf, sem_n)
    n_in.start()

    def start_in(i):
        base = i * _W
        p = i % 2
        for s in range(_SUB):
            pltpu.make_async_copy(
                emb_in.at[pl.ds(base + s * _SS, _SS)],
                bufs.at[p, pl.ds(s * _SS, _SS)],
                sem_in.at[p]).start()

    def wait_in(i):
        p = i % 2
        pltpu.make_async_copy(
            emb_in.at[pl.ds(i * _W, _W)], bufs.at[p], sem_in.at[p]).wait()

    def start_out(i):
        base = i * _W
        p = i % 2
        for s in range(_SUB):
            pltpu.make_async_copy(
                bufs.at[p, pl.ds(s * _SS, _SS)],
                emb_out.at[pl.ds(base + s * _SS, _SS)],
                sem_out.at[p]).start()

    def wait_out(i):
        p = i % 2
        pltpu.make_async_copy(
            bufs.at[p], emb_out.at[pl.ds(i * _W, _W)], sem_out.at[p]).wait()

    start_in(0)
    start_in(1)

    n_in.wait()
    row_cp = pltpu.make_async_copy(nbuf.at[pl.ds(0, 128)], row_out, sem_n)
    row_cp.start()

    for i in range(_NW):
        wait_in(i)
        start_out(i)
        if i + 2 < _NW:
            wait_out(i)
            start_in(i + 2)

    ur = tok // 64
    uc = tok % 64
    lane64 = lax.broadcasted_iota(jnp.int32, (1, 64), 1)
    u_in.wait()
    urow = ubuf[pl.ds(ur, 1), :]
    ubuf[pl.ds(ur, 1), :] = urow + (lane64 == uc).astype(jnp.float32)
    u_out = pltpu.make_async_copy(ubuf, usage_out, sem_u)
    u_out.start()

    c_in.wait()
    crow = cbuf[pl.ds(ur, 1), :]
    cbuf[pl.ds(ur, 1), :] = jnp.where(lane64 == uc, jnp.float32(jnp.inf),
                                      crow)
    c_out = pltpu.make_async_copy(cbuf, ctime_out, sem_c)
    c_out.start()

    wait_out(_NW - 2)
    wait_out(_NW - 1)

    a = (exp_row * 64 // 512) * 512
    rmw_in = pltpu.make_async_copy(emb_out.at[pl.ds(a, 512)], tbuf, sem_t)
    rmw_in.start()
    rmw_in.wait()
    blk = lax.broadcasted_iota(jnp.int32, (512,), 0) // 64
    hit = blk == (exp_row * 64 - a) // 64
    tbuf[...] = jnp.where(hit, nbuf[...], tbuf[...])
    row_w = pltpu.make_async_copy(tbuf, emb_out.at[pl.ds(a, 512)], sem_t)
    row_w.start()

    row_cp.wait()
    u_out.wait()
    c_out.wait()
    row_w.wait()


def kernel(token_usage, token_creation_time, expanded_embeddings,
           new_embedding, new_token_id):
    idx = jnp.asarray(new_token_id, jnp.int32).reshape(1)
    n_rows, dim = expanded_embeddings.shape
    emb_flat = expanded_embeddings.reshape(-1)
    usage2 = token_usage.reshape(-1, 64)
    ctime2 = token_creation_time.reshape(-1, 64)

    expanded, usage, ctime, row = pl.pallas_call(
        _body,
        in_specs=[
            pl.BlockSpec(memory_space=pltpu.SMEM),
            pl.BlockSpec(memory_space=pl.ANY),
            pl.BlockSpec(memory_space=pl.ANY),
            pl.BlockSpec(memory_space=pl.ANY),
            pl.BlockSpec(memory_space=pl.ANY),
        ],
        out_specs=[
            pl.BlockSpec(memory_space=pl.ANY),
            pl.BlockSpec(memory_space=pl.ANY),
            pl.BlockSpec(memory_space=pl.ANY),
            pl.BlockSpec(memory_space=pl.ANY),
        ],
        out_shape=[
            jax.ShapeDtypeStruct((_E,), jnp.float32),
            jax.ShapeDtypeStruct(usage2.shape, jnp.float32),
            jax.ShapeDtypeStruct(ctime2.shape, jnp.float32),
            jax.ShapeDtypeStruct((128,), jnp.float32),
        ],
        scratch_shapes=[
            pltpu.VMEM((2, _W), jnp.float32),
            pltpu.VMEM(usage2.shape, jnp.float32),
            pltpu.VMEM(ctime2.shape, jnp.float32),
            pltpu.VMEM((512,), jnp.float32),
            pltpu.VMEM((512,), jnp.float32),
            pltpu.SemaphoreType.DMA((2,)),
            pltpu.SemaphoreType.DMA((2,)),
            pltpu.SemaphoreType.DMA,
            pltpu.SemaphoreType.DMA,
            pltpu.SemaphoreType.DMA,
            pltpu.SemaphoreType.DMA,
        ],
    )(idx, emb_flat, usage2, ctime2, jnp.tile(new_embedding, 512 // dim))
    return (row[:dim], expanded.reshape(n_rows, dim), usage.reshape(-1),
            ctime.reshape(-1))


# speedup vs baseline: 2.0367x; 1.9794x over previous
import jax
import jax.numpy as jnp
from jax import lax
from jax.experimental import pallas as pl
from jax.experimental.pallas import tpu as pltpu

_INITIAL_VOCAB = 100000


def _body(idx_smem, emb_in, usage_in, ctime_in, nemb_in,
          emb_out, usage_out, ctime_out, row_out,
          nbuf, tbuf, ubuf, cbuf,
          sem_n, sem_t, sem_u, sem_c):
    tok = idx_smem[0]
    exp_row = tok - _INITIAL_VOCAB

    n_in = pltpu.make_async_copy(nemb_in, nbuf, sem_n)
    n_in.start()

    ar = (exp_row // 8) * 8
    e_in = pltpu.make_async_copy(emb_out.at[pl.ds(ar, 8)], tbuf, sem_t)
    e_in.start()
    au = (tok // 512) * 512
    u_in = pltpu.make_async_copy(usage_out.at[pl.ds(au, 512)], ubuf, sem_u)
    u_in.start()
    c_in = pltpu.make_async_copy(ctime_out.at[pl.ds(au, 512)], cbuf, sem_c)
    c_in.start()

    n_in.wait()
    row_cp = pltpu.make_async_copy(nbuf.at[pl.ds(0, 2)], row_out, sem_n)
    row_cp.start()

    e_in.wait()
    sub = lax.broadcasted_iota(jnp.int32, (8, 64), 0)
    tbuf[...] = jnp.where(sub == exp_row - ar, nbuf[...], tbuf[...])
    e_out = pltpu.make_async_copy(tbuf, emb_out.at[pl.ds(ar, 8)], sem_t)
    e_out.start()

    u_in.wait()
    lane = lax.broadcasted_iota(jnp.int32, (512,), 0)
    ubuf[...] = ubuf[...] + (lane == tok - au).astype(jnp.float32)
    u_out = pltpu.make_async_copy(ubuf, usage_out.at[pl.ds(au, 512)], sem_u)
    u_out.start()

    c_in.wait()
    cbuf[...] = jnp.where(lane == tok - au, jnp.float32(jnp.inf), cbuf[...])
    c_out = pltpu.make_async_copy(cbuf, ctime_out.at[pl.ds(au, 512)], sem_c)
    c_out.start()

    row_cp.wait()
    e_out.wait()
    u_out.wait()
    c_out.wait()


def kernel(token_usage, token_creation_time, expanded_embeddings,
           new_embedding, new_token_id):
    idx = jnp.asarray(new_token_id, jnp.int32).reshape(1)
    n_rows, dim = expanded_embeddings.shape

    expanded, usage, ctime, row = pl.pallas_call(
        _body,
        in_specs=[
            pl.BlockSpec(memory_space=pltpu.SMEM),
            pl.BlockSpec(memory_space=pl.ANY),
            pl.BlockSpec(memory_space=pl.ANY),
            pl.BlockSpec(memory_space=pl.ANY),
            pl.BlockSpec(memory_space=pl.ANY),
        ],
        out_specs=[
            pl.BlockSpec(memory_space=pl.ANY),
            pl.BlockSpec(memory_space=pl.ANY),
            pl.BlockSpec(memory_space=pl.ANY),
            pl.BlockSpec(memory_space=pl.ANY),
        ],
        out_shape=[
            jax.ShapeDtypeStruct((n_rows, dim), jnp.float32),
            jax.ShapeDtypeStruct(token_usage.shape, jnp.float32),
            jax.ShapeDtypeStruct(token_creation_time.shape, jnp.float32),
            jax.ShapeDtypeStruct((2, 64), jnp.float32),
        ],
        input_output_aliases={1: 0, 2: 1, 3: 2},
        scratch_shapes=[
            pltpu.VMEM((8, 64), jnp.float32),
            pltpu.VMEM((8, 64), jnp.float32),
            pltpu.VMEM((512,), jnp.float32),
            pltpu.VMEM((512,), jnp.float32),
            pltpu.SemaphoreType.DMA,
            pltpu.SemaphoreType.DMA,
            pltpu.SemaphoreType.DMA,
            pltpu.SemaphoreType.DMA,
        ],
    )(idx, expanded_embeddings, token_usage, token_creation_time,
      jnp.tile(new_embedding, 8).reshape(8, dim))
    return (row.reshape(-1)[:dim], expanded, usage, ctime)
